# baseline (speedup 1.0000x reference)
import functools

import jax
import jax.numpy as jnp
from jax import lax
from jax.experimental import pallas as pl
from jax.experimental.pallas import tpu as pltpu
from jax.experimental.pallas import tpu_sc as plsc

N = 100000
K = 16
D = 128
T = 512
NC = 2
NS = 16
NW = NC * NS
CHUNK = N // NW
TB = 125
NT = CHUNK // TB


def _body(x_hbm, w_hbm, idx_hbm, tab_hbm, out_hbm, tab_v, x_v, w_v, idx_v):
    wid = lax.axis_index("s") * NC + lax.axis_index("c")
    chunk = wid * CHUNK
    pltpu.sync_copy(tab_hbm, tab_v)

    def tile_body(t, carry):
        base = chunk + t * TB
        pltpu.sync_copy(x_hbm.at[pl.ds(base, TB)], x_v)
        pltpu.sync_copy(w_hbm.at[pl.ds(base, TB)], w_v)
        pltpu.sync_copy(idx_hbm.at[pl.ds(base, TB)], idx_v)

        def node_body(n, carry2):
            ivec = idx_v[n, :]
            wvec = w_v[n, :]
            idxs = [ivec[k] for k in range(K)]
            ws = [wvec[k] for k in range(K)]
            for db in range(D // 16):
                sl = pl.ds(db * 16, 16)
                acc = x_v[n, sl]
                for k in range(K):
                    acc = acc + ws[k] * tab_v[idxs[k], sl]
                x_v[n, sl] = acc
            return carry2

        lax.fori_loop(0, TB, node_body, 0)
        pltpu.sync_copy(x_v, out_hbm.at[pl.ds(base, TB)])
        return carry

    lax.fori_loop(0, NT, tile_body, 0)


def kernel(x, degree_weights, degree_indices, degree_table):
    idx = degree_indices.astype(jnp.int32)
    mesh = plsc.VectorSubcoreMesh(core_axis_name="c", subcore_axis_name="s")
    f = functools.partial(
        pl.kernel,
        out_type=jax.ShapeDtypeStruct((N, D), jnp.float32),
        mesh=mesh,
        compiler_params=pltpu.CompilerParams(use_tc_tiling_on_sc=False),
        scratch_types=[
            pltpu.VMEM((T, D), jnp.float32),
            pltpu.VMEM((TB, D), jnp.float32),
            pltpu.VMEM((TB, K), jnp.float32),
            pltpu.VMEM((TB, K), jnp.int32),
        ],
    )(_body)
    return f(x, degree_weights, idx, degree_table)


# speedup vs baseline: 2.2890x; 2.2890x over previous
import functools

import jax
import jax.numpy as jnp
from jax import lax
from jax.experimental import pallas as pl
from jax.experimental.pallas import tpu as pltpu
from jax.experimental.pallas import tpu_sc as plsc

N = 100000
K = 16
D = 128
T = 512
NC = 2
NS = 16
NW = NC * NS
CHUNK = N // NW
TB = 125
NT = CHUNK // TB
G = D // 32


def _body(x_hbm, w_hbm, idx_hbm, tab_hbm, out_hbm, tab_v, x_v, w_v, idx_v):
    wid = lax.axis_index("s") * NC + lax.axis_index("c")
    chunk = wid * CHUNK
    pltpu.sync_copy(tab_hbm, tab_v)

    def tile_body(t, carry):
        base = chunk + t * TB
        pltpu.sync_copy(x_hbm.at[pl.ds(base, TB)], x_v)
        pltpu.sync_copy(w_hbm.at[pl.ds(base, TB)], w_v)
        pltpu.sync_copy(idx_hbm.at[pl.ds(base, TB)], idx_v)

        @plsc.parallel_loop(0, TB, unroll=2)
        def node_body(n):
            ivec = idx_v[n, :]
            wvec = w_v[n, :]
            idxs = [ivec[k] for k in range(K)]
            wks = []
            for k in range(K):
                wb = jnp.broadcast_to(wvec[k], (16,))
                wks.append(plsc.pack(wb, wb, format=plsc.PackFormat.INTERLEAVED))
            for g in range(G):
                sl = pl.ds(g * 32, 32)
                acc0 = wks[0] * tab_v[idxs[0], sl]
                acc1 = wks[1] * tab_v[idxs[1], sl]
                for k in range(2, K, 2):
                    acc0 = acc0 + wks[k] * tab_v[idxs[k], sl]
                    acc1 = acc1 + wks[k + 1] * tab_v[idxs[k + 1], sl]
                a, b = plsc.unpack(acc0 + acc1,
                                   format=plsc.PackFormat.INTERLEAVED)
                sa = pl.ds(g * 32, 16)
                sb = pl.ds(g * 32 + 16, 16)
                x_v[n, sa] = x_v[n, sa] + a
                x_v[n, sb] = x_v[n, sb] + b

        pltpu.sync_copy(x_v, out_hbm.at[pl.ds(base, TB)])
        return carry

    lax.fori_loop(0, NT, tile_body, 0)


def kernel(x, degree_weights, degree_indices, degree_table):
    idx = degree_indices.astype(jnp.int32)
    tabp = (degree_table.reshape(T, G, 2, 16)
            .transpose(0, 1, 3, 2)
            .reshape(T, D)
            .astype(jnp.bfloat16))
    mesh = plsc.VectorSubcoreMesh(core_axis_name="c", subcore_axis_name="s")
    f = functools.partial(
        pl.kernel,
        out_type=jax.ShapeDtypeStruct((N, D), jnp.float32),
        mesh=mesh,
        compiler_params=pltpu.CompilerParams(
            use_tc_tiling_on_sc=False, needs_layout_passes=False),
        scratch_types=[
            pltpu.VMEM((T, D), jnp.bfloat16),
            pltpu.VMEM((TB, D), jnp.float32),
            pltpu.VMEM((TB, K), jnp.float32),
            pltpu.VMEM((TB, K), jnp.int32),
        ],
    )(_body)
    return f(x, degree_weights, idx, tabp)


# speedup vs baseline: 2.7251x; 1.1905x over previous
import functools

import jax
import jax.numpy as jnp
from jax import lax
from jax.experimental import pallas as pl
from jax.experimental.pallas import tpu as pltpu
from jax.experimental.pallas import tpu_sc as plsc

N = 100000
K = 16
D = 128
T = 512
NC = 2
NS = 16
NW = NC * NS
CHUNK = N // NW
TB = 125
NT = CHUNK // TB
G = D // 32
H1 = 62


def _body(x_hbm, w_hbm, idx_hbm, tab_hbm, out_hbm,
          tab_v, x_v, w_v, idx_v, in0, in1, out0, out1):
    wid = lax.axis_index("s") * NC + lax.axis_index("c")
    chunk = wid * CHUNK
    pltpu.sync_copy(tab_hbm, tab_v)
    in_sems = (in0, in1)
    out_sems = (out0, out1)

    def in_descs(t, b):
        src = pl.ds(chunk + t * TB, TB)
        dst = pl.ds(b * TB, TB)
        return (
            pltpu.make_async_copy(x_hbm.at[src], x_v.at[dst], in_sems[b]),
            pltpu.make_async_copy(w_hbm.at[src], w_v.at[dst], in_sems[b]),
            pltpu.make_async_copy(idx_hbm.at[src], idx_v.at[dst], in_sems[b]),
        )

    def out_desc(t, b):
        return pltpu.make_async_copy(
            x_v.at[pl.ds(b * TB, TB)],
            out_hbm.at[pl.ds(chunk + t * TB, TB)],
            out_sems[b])

    def compute(b, lo, hi):
        @plsc.parallel_loop(lo, hi, unroll=2)
        def node_body(n):
            nn = n + b * TB
            ivec = idx_v[nn, :]
            wvec = w_v[nn, :]
            idxs = [ivec[k] for k in range(K)]
            wks = [plsc.bitcast(jnp.broadcast_to(wvec[k], (16,)),
                                jnp.bfloat16) for k in range(K)]
            for g in range(G):
                sl = pl.ds(g * 32, 32)
                acc0 = wks[0] * tab_v[idxs[0], sl]
                acc1 = wks[1] * tab_v[idxs[1], sl]
                for k in range(2, K, 2):
                    acc0 = acc0 + wks[k] * tab_v[idxs[k], sl]
                    acc1 = acc1 + wks[k + 1] * tab_v[idxs[k + 1], sl]
                a, bb = plsc.unpack(acc0 + acc1,
                                    format=plsc.PackFormat.INTERLEAVED)
                sa = pl.ds(g * 32, 16)
                sb = pl.ds(g * 32 + 16, 16)
                x_v[nn, sa] = x_v[nn, sa] + a
                x_v[nn, sb] = x_v[nn, sb] + bb

    def process(t, b):
        nb = 1 - b
        for dsc in in_descs(t, b):
            dsc.wait()
        compute(b, 0, H1)
        @pl.when(t >= 1)
        def _():
            out_desc(t - 1, nb).wait()

        @pl.when(t + 1 < NT)
        def _():
            for dsc in in_descs(t + 1, nb):
                dsc.start()

        compute(b, H1, TB)
        out_desc(t, b).start()

    for dsc in in_descs(0, 0):
        dsc.start()

    def pair(p, carry):
        process(2 * p, 0)
        process(2 * p + 1, 1)
        return carry

    lax.fori_loop(0, NT // 2, pair, 0)
    process(NT - 1, 0)
    out_desc(NT - 1, 0).wait()


def kernel(x, degree_weights, degree_indices, degree_table):
    idx = degree_indices.astype(jnp.int32)
    wb = jax.lax.bitcast_convert_type(
        degree_weights.astype(jnp.bfloat16), jnp.uint16).astype(jnp.uint32)
    wp = jax.lax.bitcast_convert_type(wb | (wb << 16), jnp.int32)
    tabp = (degree_table.reshape(T, G, 2, 16)
            .transpose(0, 1, 3, 2)
            .reshape(T, D)
            .astype(jnp.bfloat16))
    mesh = plsc.VectorSubcoreMesh(core_axis_name="c", subcore_axis_name="s")
    f = functools.partial(
        pl.kernel,
        out_type=jax.ShapeDtypeStruct((N, D), jnp.float32),
        mesh=mesh,
        compiler_params=pltpu.CompilerParams(
            use_tc_tiling_on_sc=False, needs_layout_passes=False),
        scratch_types=[
            pltpu.VMEM((T, D), jnp.bfloat16),
            pltpu.VMEM((2 * TB, D), jnp.float32),
            pltpu.VMEM((2 * TB, K), jnp.int32),
            pltpu.VMEM((2 * TB, K), jnp.int32),
            pltpu.SemaphoreType.DMA,
            pltpu.SemaphoreType.DMA,
            pltpu.SemaphoreType.DMA,
            pltpu.SemaphoreType.DMA,
        ],
    )(_body)
    return f(x, wp, idx, tabp)
